# Initial kernel scaffold; baseline (speedup 1.0000x reference)
import jax
import jax.numpy as jnp
from jax.experimental import pallas as pl


def kernel(in_triple, ent_emb, ent_covar, rel_emb, rel_covar):
    raise NotImplementedError("write your pallas kernel here")


# speedup vs baseline: 2.3756x; 2.3756x over previous
import functools

import jax
import jax.numpy as jnp
from jax import lax
from jax.experimental import pallas as pl
from jax.experimental.pallas import tpu as pltpu
from jax.experimental.pallas import tpu_sc as plsc

_ENT_SIZE = 100000
_EMB_DIM = 128
_BATCH = 16384

_NC = 2
_NS = 16
_NW = _NC * _NS
_BPW = _BATCH // _NW
_C = 64
_NCHUNK = _BPW // _C
_NGRP = _C // 16


def _sc_body(head_hbm, rel_hbm, tail_hbm,
             ent_emb, ent_covar, rel_emb, rel_covar,
             out_hbm,
             hidx, ridx, tidx,
             hm0, hv0, tm0, tv0, rm0, rv0,
             hm1, hv1, tm1, tv1, rm1, rv1,
             out_v, sem0, sem1):
    wid = lax.axis_index("s") * _NC + lax.axis_index("c")
    base = wid * _BPW

    pltpu.sync_copy(head_hbm.at[pl.ds(base, _BPW)], hidx)
    pltpu.sync_copy(rel_hbm.at[pl.ds(base, _BPW)], ridx)
    pltpu.sync_copy(tail_hbm.at[pl.ds(base, _BPW)], tidx)

    sems = (sem0, sem1)
    bufsets = ((hm0, hv0, tm0, tv0, rm0, rv0),
               (hm1, hv1, tm1, tv1, rm1, rv1))

    def fire(c):
        s = c % 2
        off = c * _C
        hi = hidx.at[pl.ds(off, _C)]
        ri = ridx.at[pl.ds(off, _C)]
        ti = tidx.at[pl.ds(off, _C)]
        srcs = (ent_emb.at[hi], ent_covar.at[hi],
                ent_emb.at[ti], ent_covar.at[ti],
                rel_emb.at[ri], rel_covar.at[ri])
        return [pltpu.async_copy(src, buf, sems[s])
                for src, buf in zip(srcs, bufsets[s])]

    lanes = lax.iota(jnp.int32, 16)
    rot = lanes * 9

    descs = fire(0)
    for c in range(_NCHUNK):
        if c + 1 < _NCHUNK:
            next_descs = fire(c + 1)
        for d in descs:
            d.wait()
        hm, hv, tm, tv, rm, rv = bufsets[c % 2]

        def body(j, accs):
            col = (j + rot) & (_EMB_DIM - 1)
            new = []
            for g in range(_NGRP):
                rows = lanes + g * 16
                a = plsc.load_gather(hm, [rows, col])
                b = plsc.load_gather(hv, [rows, col])
                cm = plsc.load_gather(tm, [rows, col])
                cv = plsc.load_gather(tv, [rows, col])
                dm = plsc.load_gather(rm, [rows, col])
                dv = plsc.load_gather(rv, [rows, col])
                ev = cv + b
                d_ = dm - (cm - a)
                d2 = d_ * d_
                new.append(accs[g] + (ev + d2) / dv + (dv + d2) / ev)
            return tuple(new)

        zero = jnp.zeros((16,), jnp.float32)
        accs = lax.fori_loop(0, _EMB_DIM, body, (zero,) * _NGRP)
        for g in range(_NGRP):
            out_v[pl.ds(c * _C + g * 16, 16)] = (accs[g] - 2.0 * _EMB_DIM) * 0.25

        if c + 1 < _NCHUNK:
            descs = next_descs

    pltpu.sync_copy(out_v, out_hbm.at[pl.ds(base, _BPW)])


_sc_kernel = functools.partial(
    pl.kernel,
    out_type=jax.ShapeDtypeStruct((_BATCH,), jnp.float32),
    mesh=plsc.VectorSubcoreMesh(core_axis_name="c", subcore_axis_name="s"),
    scratch_types=[
        pltpu.VMEM((_BPW,), jnp.int32),
        pltpu.VMEM((_BPW,), jnp.int32),
        pltpu.VMEM((_BPW,), jnp.int32),
    ] + [pltpu.VMEM((_C, _EMB_DIM), jnp.float32)] * 12 + [
        pltpu.VMEM((_BPW,), jnp.float32),
        pltpu.SemaphoreType.DMA,
        pltpu.SemaphoreType.DMA,
    ],
    compiler_params=pltpu.CompilerParams(
        use_tc_tiling_on_sc=False, needs_layout_passes=False),
)(_sc_body)


def kernel(in_triple, ent_emb, ent_covar, rel_emb, rel_covar):
    head = in_triple[:, 0]
    rel = in_triple[:, 1]
    tail = in_triple[:, 2]
    return _sc_kernel(head, rel, tail, ent_emb, ent_covar, rel_emb, rel_covar)
